# Initial kernel scaffold; baseline (speedup 1.0000x reference)
import jax
import jax.numpy as jnp
from jax.experimental import pallas as pl


def kernel(x, edge_index, W_ih1, W_hh1, b_ih1, b_hh1, W_self1, W_neigh1, bias1, W_ih2, W_hh2, b_ih2, b_hh2, W_self2, W_neigh2, bias2):
    raise NotImplementedError("write your pallas kernel here")


# speedup vs baseline: 3.3157x; 3.3157x over previous
import functools

import jax
import jax.numpy as jnp
from jax import lax
from jax.experimental import pallas as pl
from jax.experimental.pallas import tpu as pltpu
from jax.experimental.pallas import tpu_sc as plsc

N = 10000
K = 16
D = 128
H = 128

_NW = 32
_R = N * K
_RPW = _R // _NW
_C = 120
_NCH = 42
_CLAST = _RPW - (_NCH - 1) * _C

_B = 400


def _gather_rows(table, idx_grp):
    mesh = plsc.VectorSubcoreMesh(core_axis_name="c", subcore_axis_name="s")

    @functools.partial(
        pl.kernel,
        mesh=mesh,
        out_type=jax.ShapeDtypeStruct((_R, D), jnp.float32),
        scratch_types=[
            pltpu.VMEM((_NCH, _C), jnp.int32),
            pltpu.VMEM((2, _C, D), jnp.float32),
            pltpu.SemaphoreType.DMA,
            pltpu.SemaphoreType.DMA,
        ],
    )
    def k(table_hbm, idx_hbm, out_hbm, idx_v, rows_v, sem0, sem1):
        sems = (sem0, sem1)
        wid = lax.axis_index("s") * 2 + lax.axis_index("c")
        base = wid * _RPW
        pltpu.sync_copy(idx_hbm.at[wid], idx_v)

        def start(j, b):
            pltpu.make_async_copy(
                table_hbm.at[idx_v.at[j]], rows_v.at[b], sems[b]).start()

        def finish(j, b):
            pltpu.make_async_copy(
                table_hbm.at[idx_v.at[j]], rows_v.at[b], sems[b]).wait()
            pltpu.sync_copy(rows_v.at[b],
                            out_hbm.at[pl.ds(base + j * _C, _C)])

        start(0, 0)
        start(1, 1)

        def pair(p, carry):
            for b in range(2):
                j = p * 2 + b
                finish(j, b)
                start(j + 2, b)
            return carry

        lax.fori_loop(0, _NCH // 2 - 1, pair, 0)
        finish(_NCH - 2, 0)
        pltpu.make_async_copy(
            table_hbm.at[idx_v.at[_NCH - 1]], rows_v.at[1], sems[1]).wait()
        pltpu.sync_copy(
            rows_v.at[1, pl.ds(0, _CLAST)],
            out_hbm.at[pl.ds(base + (_NCH - 1) * _C, _CLAST)])

    return k(table, idx_grp)


def _sage_layer(m, xin, wihT, whhT, bvec, wselfT, wneighT, bout, act):
    def body(m_ref, x_ref, wih_ref, whh_ref, b_ref, ws_ref, wn_ref, bo_ref,
             o_ref):
        wih = wih_ref[...]
        whh = whh_ref[...]
        bias = b_ref[...]
        h = jnp.zeros((_B, H), jnp.float32)
        c = jnp.zeros((_B, H), jnp.float32)
        for t in range(K):
            g = jnp.dot(m_ref[t], wih, preferred_element_type=jnp.float32)
            g = g + jnp.dot(h, whh, preferred_element_type=jnp.float32) + bias
            gi = jax.nn.sigmoid(g[:, 0 * H:1 * H])
            gf = jax.nn.sigmoid(g[:, 1 * H:2 * H])
            gg = jnp.tanh(g[:, 2 * H:3 * H])
            go = jax.nn.sigmoid(g[:, 3 * H:4 * H])
            c = gf * c + gi * gg
            h = go * jnp.tanh(c)
        out = (jnp.dot(x_ref[...], ws_ref[...],
                       preferred_element_type=jnp.float32)
               + jnp.dot(h, wn_ref[...], preferred_element_type=jnp.float32)
               + bo_ref[...])
        o_ref[...] = act(out)

    return pl.pallas_call(
        body,
        grid=(N // _B,),
        in_specs=[
            pl.BlockSpec((K, _B, D), lambda i: (0, i, 0)),
            pl.BlockSpec((_B, D), lambda i: (i, 0)),
            pl.BlockSpec((D, 4 * H), lambda i: (0, 0)),
            pl.BlockSpec((H, 4 * H), lambda i: (0, 0)),
            pl.BlockSpec((1, 4 * H), lambda i: (0, 0)),
            pl.BlockSpec((D, H), lambda i: (0, 0)),
            pl.BlockSpec((H, H), lambda i: (0, 0)),
            pl.BlockSpec((1, H), lambda i: (0, 0)),
        ],
        out_specs=pl.BlockSpec((_B, H), lambda i: (i, 0)),
        out_shape=jax.ShapeDtypeStruct((N, H), jnp.float32),
    )(m, xin, wihT, whhT, bvec, wselfT, wneighT, bout)


def kernel(x, edge_index, W_ih1, W_hh1, b_ih1, b_hh1, W_self1, W_neigh1,
           bias1, W_ih2, W_hh2, b_ih2, b_hh2, W_self2, W_neigh2, bias2):
    src = edge_index[0]
    idx_flat = src.reshape(N, K).T.reshape(_NW, _RPW)
    pad = jnp.zeros((_NW, _NCH * _C - _RPW), jnp.int32)
    idx = jnp.concatenate([idx_flat, pad], axis=1).reshape(_NW, _NCH, _C)

    m1 = _gather_rows(x, idx).reshape(K, N, D)
    h1 = _sage_layer(
        m1, x, W_ih1.T, W_hh1.T, (b_ih1 + b_hh1).reshape(1, -1),
        W_self1.T, W_neigh1.T, bias1.reshape(1, -1), jax.nn.relu)
    m2 = _gather_rows(h1, idx).reshape(K, N, H)
    out = _sage_layer(
        m2, h1, W_ih2.T, W_hh2.T, (b_ih2 + b_hh2).reshape(1, -1),
        W_self2.T, W_neigh2.T, bias2.reshape(1, -1), jax.nn.sigmoid)
    return out


# speedup vs baseline: 4.2066x; 1.2687x over previous
import functools

import jax
import jax.numpy as jnp
from jax import lax
from jax.experimental import pallas as pl
from jax.experimental.pallas import tpu as pltpu
from jax.experimental.pallas import tpu_sc as plsc

N = 10000
K = 16
D = 128
H = 128

_NW = 32
_R = N * K
_RPW = _R // _NW
_C = 120
_NCH = 42
_CLAST = _RPW - (_NCH - 1) * _C

_B = 1000


def _gather_rows(table, idx_grp):
    mesh = plsc.VectorSubcoreMesh(core_axis_name="c", subcore_axis_name="s")

    @functools.partial(
        pl.kernel,
        mesh=mesh,
        out_type=jax.ShapeDtypeStruct((_R, D), jnp.float32),
        scratch_types=[
            pltpu.VMEM((_NCH, _C), jnp.int32),
            pltpu.VMEM((2, _C, D), jnp.float32),
            pltpu.SemaphoreType.DMA,
            pltpu.SemaphoreType.DMA,
        ],
    )
    def k(table_hbm, idx_hbm, out_hbm, idx_v, rows_v, sem0, sem1):
        sems = (sem0, sem1)
        wid = lax.axis_index("s") * 2 + lax.axis_index("c")
        base = wid * _RPW
        pltpu.sync_copy(idx_hbm.at[wid], idx_v)

        def start(j, b):
            pltpu.make_async_copy(
                table_hbm.at[idx_v.at[j]], rows_v.at[b], sems[b]).start()

        def finish(j, b):
            pltpu.make_async_copy(
                table_hbm.at[idx_v.at[j]], rows_v.at[b], sems[b]).wait()
            pltpu.sync_copy(rows_v.at[b],
                            out_hbm.at[pl.ds(base + j * _C, _C)])

        start(0, 0)
        start(1, 1)

        def pair(p, carry):
            for b in range(2):
                j = p * 2 + b
                finish(j, b)
                start(j + 2, b)
            return carry

        lax.fori_loop(0, _NCH // 2 - 1, pair, 0)
        finish(_NCH - 2, 0)
        pltpu.make_async_copy(
            table_hbm.at[idx_v.at[_NCH - 1]], rows_v.at[1], sems[1]).wait()
        pltpu.sync_copy(
            rows_v.at[1, pl.ds(0, _CLAST)],
            out_hbm.at[pl.ds(base + (_NCH - 1) * _C, _CLAST)])

    return k(table, idx_grp)


def _sage_layer(m, xin, wihT, whhT, bvec, wselfT, wneighT, bout, act):
    def body(m_ref, x_ref, wih_ref, whh_ref, b_ref, ws_ref, wn_ref, bo_ref,
             o_ref):
        wih = wih_ref[...]
        whh = whh_ref[...]
        bias = b_ref[...]
        h = jnp.zeros((_B, H), jnp.float32)
        c = jnp.zeros((_B, H), jnp.float32)
        for t in range(K):
            g = jnp.dot(m_ref[t], wih, preferred_element_type=jnp.float32)
            g = g + jnp.dot(h, whh, preferred_element_type=jnp.float32) + bias
            gi = 0.5 * jnp.tanh(g[:, 0 * H:1 * H]) + 0.5
            gf = 0.5 * jnp.tanh(g[:, 1 * H:2 * H]) + 0.5
            gg = jnp.tanh(g[:, 2 * H:3 * H])
            go = 0.5 * jnp.tanh(g[:, 3 * H:4 * H]) + 0.5
            c = gf * c + gi * gg
            h = go * jnp.tanh(c)
        out = (jnp.dot(x_ref[...], ws_ref[...],
                       preferred_element_type=jnp.float32)
               + jnp.dot(h, wn_ref[...], preferred_element_type=jnp.float32)
               + bo_ref[...])
        o_ref[...] = act(out)

    return pl.pallas_call(
        body,
        grid=(N // _B,),
        in_specs=[
            pl.BlockSpec((K, _B, D), lambda i: (0, i, 0)),
            pl.BlockSpec((_B, D), lambda i: (i, 0)),
            pl.BlockSpec((D, 4 * H), lambda i: (0, 0)),
            pl.BlockSpec((H, 4 * H), lambda i: (0, 0)),
            pl.BlockSpec((1, 4 * H), lambda i: (0, 0)),
            pl.BlockSpec((D, H), lambda i: (0, 0)),
            pl.BlockSpec((H, H), lambda i: (0, 0)),
            pl.BlockSpec((1, H), lambda i: (0, 0)),
        ],
        out_specs=pl.BlockSpec((_B, H), lambda i: (i, 0)),
        out_shape=jax.ShapeDtypeStruct((N, H), jnp.float32),
    )(m, xin, wihT, whhT, bvec, wselfT, wneighT, bout)


def kernel(x, edge_index, W_ih1, W_hh1, b_ih1, b_hh1, W_self1, W_neigh1,
           bias1, W_ih2, W_hh2, b_ih2, b_hh2, W_self2, W_neigh2, bias2):
    src = edge_index[0]
    idx_flat = src.reshape(N, K).T.reshape(_NW, _RPW)
    pad = jnp.zeros((_NW, _NCH * _C - _RPW), jnp.int32)
    idx = jnp.concatenate([idx_flat, pad], axis=1).reshape(_NW, _NCH, _C)

    s = jnp.concatenate([
        jnp.full((H,), 0.5, jnp.float32),
        jnp.full((H,), 0.5, jnp.float32),
        jnp.ones((H,), jnp.float32),
        jnp.full((H,), 0.5, jnp.float32),
    ])

    m1 = _gather_rows(x, idx).reshape(K, N, D)
    h1 = _sage_layer(
        m1, x, W_ih1.T * s, W_hh1.T * s,
        ((b_ih1 + b_hh1) * s).reshape(1, -1),
        W_self1.T, W_neigh1.T, bias1.reshape(1, -1), jax.nn.relu)
    m2 = _gather_rows(h1, idx).reshape(K, N, H)
    out = _sage_layer(
        m2, h1, W_ih2.T * s, W_hh2.T * s,
        ((b_ih2 + b_hh2) * s).reshape(1, -1),
        W_self2.T, W_neigh2.T, bias2.reshape(1, -1), jax.nn.sigmoid)
    return out


# speedup vs baseline: 4.2309x; 1.0058x over previous
import functools

import jax
import jax.numpy as jnp
from jax import lax
from jax.experimental import pallas as pl
from jax.experimental.pallas import tpu as pltpu
from jax.experimental.pallas import tpu_sc as plsc

N = 10000
K = 16
D = 128
H = 128

_NW = 32
_R = N * K
_RPW = _R // _NW
_C = 120
_NCH = 42
_CLAST = _RPW - (_NCH - 1) * _C

_B = 1000


_NBUF = 6
_DEPTH = 3


def _gather_rows(table, idx_grp):
    mesh = plsc.VectorSubcoreMesh(core_axis_name="c", subcore_axis_name="s")

    @functools.partial(
        pl.kernel,
        mesh=mesh,
        out_type=jax.ShapeDtypeStruct((_R, D), jnp.float32),
        scratch_types=[
            pltpu.VMEM((_NCH, _C), jnp.int32),
            pltpu.VMEM((_NBUF, _C, D), jnp.float32),
        ] + [pltpu.SemaphoreType.DMA] * (2 * _NBUF),
    )
    def k(table_hbm, idx_hbm, out_hbm, idx_v, rows_v, *sems):
        gs, ws = sems[:_NBUF], sems[_NBUF:]
        wid = lax.axis_index("s") * 2 + lax.axis_index("c")
        base = wid * _RPW
        pltpu.sync_copy(idx_hbm.at[wid], idx_v)

        def gstart(j, b):
            pltpu.make_async_copy(
                table_hbm.at[idx_v.at[j]], rows_v.at[b], gs[b]).start()

        def gwait(j, b):
            pltpu.make_async_copy(
                table_hbm.at[idx_v.at[j]], rows_v.at[b], gs[b]).wait()

        def wdesc(j, b, n):
            return pltpu.make_async_copy(
                rows_v.at[b, pl.ds(0, n)],
                out_hbm.at[pl.ds(base + j * _C, n)], ws[b])

        for b in range(_DEPTH):
            gstart(b, b)

        def group(g, carry):
            for b in range(_NBUF):
                j = g * _NBUF + b
                gwait(j, b)
                if b == _NBUF - 1:
                    @pl.when(g < _NCH // _NBUF - 1)
                    def _():
                        wdesc(j, b, _C).start()

                    @pl.when(g == _NCH // _NBUF - 1)
                    def _():
                        wdesc(j, b, _CLAST).start()
                else:
                    wdesc(j, b, _C).start()
                bn = (b + _DEPTH) % _NBUF
                jn = j + _DEPTH
                if b < _DEPTH:
                    @pl.when(g >= 1)
                    def _():
                        wdesc(0, bn, _C).wait()
                    gstart(jn, bn)
                else:
                    @pl.when(g < _NCH // _NBUF - 1)
                    def _():
                        wdesc(0, bn, _C).wait()
                        gstart(jn, bn)
            return carry

        lax.fori_loop(0, _NCH // _NBUF, group, 0)
        for b in range(_NBUF):
            wdesc(0, b, _C if b < _NBUF - 1 else _CLAST).wait()

    return k(table, idx_grp)


def _sage_layer(m, xin, wihT, whhT, bvec, wselfT, wneighT, bout, act):
    def body(m_ref, x_ref, wih_ref, whh_ref, b_ref, ws_ref, wn_ref, bo_ref,
             o_ref):
        wih = wih_ref[...]
        whh = whh_ref[...]
        bias = b_ref[...]
        h = jnp.zeros((_B, H), jnp.float32)
        c = jnp.zeros((_B, H), jnp.float32)
        for t in range(K):
            g = jnp.dot(m_ref[t], wih, preferred_element_type=jnp.float32)
            g = g + jnp.dot(h, whh, preferred_element_type=jnp.float32) + bias
            gi = 0.5 * jnp.tanh(g[:, 0 * H:1 * H]) + 0.5
            gf = 0.5 * jnp.tanh(g[:, 1 * H:2 * H]) + 0.5
            gg = jnp.tanh(g[:, 2 * H:3 * H])
            go = 0.5 * jnp.tanh(g[:, 3 * H:4 * H]) + 0.5
            c = gf * c + gi * gg
            h = go * jnp.tanh(c)
        out = (jnp.dot(x_ref[...], ws_ref[...],
                       preferred_element_type=jnp.float32)
               + jnp.dot(h, wn_ref[...], preferred_element_type=jnp.float32)
               + bo_ref[...])
        o_ref[...] = act(out)

    return pl.pallas_call(
        body,
        grid=(N // _B,),
        in_specs=[
            pl.BlockSpec((K, _B, D), lambda i: (0, i, 0)),
            pl.BlockSpec((_B, D), lambda i: (i, 0)),
            pl.BlockSpec((D, 4 * H), lambda i: (0, 0)),
            pl.BlockSpec((H, 4 * H), lambda i: (0, 0)),
            pl.BlockSpec((1, 4 * H), lambda i: (0, 0)),
            pl.BlockSpec((D, H), lambda i: (0, 0)),
            pl.BlockSpec((H, H), lambda i: (0, 0)),
            pl.BlockSpec((1, H), lambda i: (0, 0)),
        ],
        out_specs=pl.BlockSpec((_B, H), lambda i: (i, 0)),
        out_shape=jax.ShapeDtypeStruct((N, H), jnp.float32),
    )(m, xin, wihT, whhT, bvec, wselfT, wneighT, bout)


def kernel(x, edge_index, W_ih1, W_hh1, b_ih1, b_hh1, W_self1, W_neigh1,
           bias1, W_ih2, W_hh2, b_ih2, b_hh2, W_self2, W_neigh2, bias2):
    src = edge_index[0]
    idx_flat = src.reshape(N, K).T.reshape(_NW, _RPW)
    pad = jnp.zeros((_NW, _NCH * _C - _RPW), jnp.int32)
    idx = jnp.concatenate([idx_flat, pad], axis=1).reshape(_NW, _NCH, _C)

    s = jnp.concatenate([
        jnp.full((H,), 0.5, jnp.float32),
        jnp.full((H,), 0.5, jnp.float32),
        jnp.ones((H,), jnp.float32),
        jnp.full((H,), 0.5, jnp.float32),
    ])

    m1 = _gather_rows(x, idx).reshape(K, N, D)
    h1 = _sage_layer(
        m1, x, W_ih1.T * s, W_hh1.T * s,
        ((b_ih1 + b_hh1) * s).reshape(1, -1),
        W_self1.T, W_neigh1.T, bias1.reshape(1, -1), jax.nn.relu)
    m2 = _gather_rows(h1, idx).reshape(K, N, H)
    out = _sage_layer(
        m2, h1, W_ih2.T * s, W_hh2.T * s,
        ((b_ih2 + b_hh2) * s).reshape(1, -1),
        W_self2.T, W_neigh2.T, bias2.reshape(1, -1), jax.nn.sigmoid)
    return out
